# baseline (speedup 1.0000x reference)
import jax

jax.config.update("jax_enable_x64", True)

import jax.numpy as jnp
import numpy as np
from jax import lax
from jax.experimental import pallas as pl

NUM_CLASSES = 1000
N = 16384
COLS_PER_BLOCK = 2048
_I32_ZERO = np.int32(0)


def _onehot_t_block(x_ref, out_ref):
    iota = lax.broadcasted_iota(
        jnp.int32, (NUM_CLASSES, COLS_PER_BLOCK), 0
    )
    cmp = iota == x_ref[0, :][None, :]
    out_ref[...] = cmp.astype(jnp.int32)


def kernel(x):
    x32 = x.astype(jnp.int32).reshape(1, N)
    grid = (N // COLS_PER_BLOCK,)
    lo_t = pl.pallas_call(
        _onehot_t_block,
        grid=grid,
        in_specs=[pl.BlockSpec((1, COLS_PER_BLOCK), lambda g: (_I32_ZERO, g))],
        out_specs=pl.BlockSpec(
            (NUM_CLASSES, COLS_PER_BLOCK), lambda g: (_I32_ZERO, g)
        ),
        out_shape=jax.ShapeDtypeStruct((NUM_CLASSES, N), jnp.int32),
    )(x32)
    z = jnp.zeros((N, NUM_CLASSES), jnp.uint32)
    return z.at[0, 0].set(lo_t[0, 0].astype(jnp.uint32)).astype(jnp.int64)


# speedup vs baseline: 1.1091x; 1.1091x over previous
import jax

jax.config.update("jax_enable_x64", True)

import jax.numpy as jnp
import numpy as np
from jax import lax
from jax.experimental import pallas as pl

NUM_CLASSES = 1000
N = 16384
COLS_PER_BLOCK = 4096
_I32_ZERO = np.int32(0)


def _onehot_t_block(x_ref, out_ref):
    iota = lax.broadcasted_iota(
        jnp.int32, (NUM_CLASSES, COLS_PER_BLOCK), 0
    )
    cmp = iota == x_ref[0, :][None, :]
    out_ref[...] = cmp.astype(jnp.int32)


def kernel(x):
    x32 = x.astype(jnp.int32).reshape(1, N)
    grid = (N // COLS_PER_BLOCK,)
    lo_t = pl.pallas_call(
        _onehot_t_block,
        grid=grid,
        in_specs=[pl.BlockSpec((1, COLS_PER_BLOCK), lambda g: (_I32_ZERO, g))],
        out_specs=pl.BlockSpec(
            (NUM_CLASSES, COLS_PER_BLOCK), lambda g: (_I32_ZERO, g)
        ),
        out_shape=jax.ShapeDtypeStruct((NUM_CLASSES, N), jnp.int32),
    )(x32)
    return lo_t.T.astype(jnp.int64)


# speedup vs baseline: 1.1529x; 1.0395x over previous
import jax

jax.config.update("jax_enable_x64", True)

import jax.numpy as jnp
import numpy as np
from jax import lax
from jax.experimental import pallas as pl

NUM_CLASSES = 1000
N = 16384
COLS_PER_BLOCK = 4096
_I32_ZERO = np.int32(0)


def _onehot_t_block(x_ref, out_ref):
    iota = lax.broadcasted_iota(
        jnp.int32, (NUM_CLASSES, COLS_PER_BLOCK), 0
    )
    cmp = iota == x_ref[0, :][None, :]
    out_ref[...] = cmp.astype(jnp.uint32)


def kernel(x):
    x32 = x.astype(jnp.int32).reshape(1, N)
    grid = (N // COLS_PER_BLOCK,)
    lo_t = pl.pallas_call(
        _onehot_t_block,
        grid=grid,
        in_specs=[pl.BlockSpec((1, COLS_PER_BLOCK), lambda g: (_I32_ZERO, g))],
        out_specs=pl.BlockSpec(
            (NUM_CLASSES, COLS_PER_BLOCK), lambda g: (_I32_ZERO, g)
        ),
        out_shape=jax.ShapeDtypeStruct((NUM_CLASSES, N), jnp.uint32),
    )(x32)
    return lo_t.T.astype(jnp.int64)
